# baseline (speedup 1.0000x reference)
import jax
import jax.numpy as jnp
from jax import lax
from jax.experimental import pallas as pl
from jax.experimental.pallas import tpu as pltpu
from jax.experimental.pallas import tpu_sc as plsc

_VOCAB = 100000
_EMBED = 64
_BATCH = 1024
_CTX = 50
_NC = 2
_NS = 16
_NW = _NC * _NS
_FIRE = 8


def _sc_pool(input_ids, emb):
    batch = input_ids.shape[0]
    bpw = batch // _NW

    def body(ids_hbm, emb_hbm, out_hbm, idx_v, rows_v, out_v, sem):
        wid = lax.axis_index("s") * _NC + lax.axis_index("c")
        base = wid * bpw
        pltpu.sync_copy(ids_hbm.at[pl.ds(base, bpw), :], idx_v)

        for g in range(pl.cdiv(bpw, _FIRE)):
            cps = []
            for i in range(min(_FIRE, bpw - g * _FIRE)):
                b = g * _FIRE + i
                cps.append(
                    pltpu.async_copy(emb_hbm.at[idx_v.at[b]], rows_v.at[b],
                                     sem)
                )
            for cp in cps:
                cp.wait()

        def bbody(b, carry):
            accs = [rows_v[b, 0, pl.ds(16 * d, 16)] for d in range(4)]
            for c in range(1, _CTX):
                for d in range(4):
                    accs[d] = accs[d] + rows_v[b, c, pl.ds(16 * d, 16)]
            for d in range(4):
                out_v[b, pl.ds(16 * d, 16)] = accs[d] * (1.0 / _CTX)
            return carry

        lax.fori_loop(0, bpw, bbody, 0)
        pltpu.sync_copy(out_v, out_hbm.at[pl.ds(base, bpw), :])

    mesh = plsc.VectorSubcoreMesh(core_axis_name="c", subcore_axis_name="s")
    return pl.kernel(
        body,
        out_type=jax.ShapeDtypeStruct((batch, _EMBED), jnp.float32),
        mesh=mesh,
        compiler_params=pltpu.CompilerParams(use_tc_tiling_on_sc=False),
        scratch_types=[
            pltpu.VMEM((bpw, _CTX), jnp.int32),
            pltpu.VMEM((bpw, _CTX, _EMBED), jnp.float32),
            pltpu.VMEM((bpw, _EMBED), jnp.float32),
            pltpu.SemaphoreType.DMA,
        ],
    )(input_ids, emb)


_BSUB = 32
_NOBUF = 3
_CROWS = 16
_NCHUNK = _BSUB // _CROWS


def _tc_mlp(pooled, W1, b1, wft_bf16, bf):
    batch = pooled.shape[0]
    nstep = batch // _BSUB

    def body(pooled_ref, w1_ref, b1_ref, wft_ref, bf_ref, out_hbm,
             obuf, sems):
        i = pl.program_id(0)
        slot = lax.rem(i, _NOBUF)

        @pl.when(i >= _NOBUF)
        def _():
            for c in range(_NCHUNK):
                pltpu.make_async_copy(
                    obuf.at[slot, pl.ds(c * _CROWS, _CROWS), :],
                    out_hbm.at[
                        pl.ds((i - _NOBUF) * _BSUB + c * _CROWS, _CROWS), :],
                    sems.at[slot],
                ).wait()

        hidden = jnp.maximum(
            lax.dot_general(
                pooled_ref[...].astype(jnp.bfloat16),
                w1_ref[...].astype(jnp.bfloat16),
                (((1,), (1,)), ((), ())),
                preferred_element_type=jnp.float32,
            ) + b1_ref[...],
            0.0,
        )
        obuf[slot] = (lax.dot_general(
            hidden.astype(jnp.bfloat16), wft_ref[...],
            (((1,), (0,)), ((), ())),
            preferred_element_type=jnp.float32,
        ) + bf_ref[...]).astype(jnp.bfloat16)

        for c in range(_NCHUNK):
            pltpu.make_async_copy(
                obuf.at[slot, pl.ds(c * _CROWS, _CROWS), :],
                out_hbm.at[pl.ds(i * _BSUB + c * _CROWS, _CROWS), :],
                sems.at[slot],
            ).start()

        @pl.when(i == nstep - 1)
        def _():
            for k in range(_NOBUF):
                s = (slot - k) % _NOBUF
                for c in range(_NCHUNK):
                    pltpu.make_async_copy(
                        obuf.at[s, pl.ds(c * _CROWS, _CROWS), :],
                        out_hbm.at[
                            pl.ds((i - k) * _BSUB + c * _CROWS, _CROWS), :],
                        sems.at[s],
                    ).wait()

    return pl.pallas_call(
        body,
        grid=(nstep,),
        in_specs=[
            pl.BlockSpec((_BSUB, _EMBED), lambda i: (i, 0)),
            pl.BlockSpec((_EMBED // 2, _EMBED), lambda i: (0, 0)),
            pl.BlockSpec((1, _EMBED // 2), lambda i: (0, 0)),
            pl.BlockSpec((_EMBED // 2, _VOCAB), lambda i: (0, 0)),
            pl.BlockSpec((1, _VOCAB), lambda i: (0, 0)),
        ],
        out_specs=pl.BlockSpec(memory_space=pl.ANY),
        out_shape=jax.ShapeDtypeStruct((batch, _VOCAB), jnp.bfloat16),
        scratch_shapes=[
            pltpu.VMEM((_NOBUF, _BSUB, _VOCAB), jnp.bfloat16),
            pltpu.SemaphoreType.DMA((_NOBUF,)),
        ],
        compiler_params=pltpu.CompilerParams(
            vmem_limit_bytes=60_000_000,
        ),
    )(pooled, W1, b1.reshape(1, _EMBED // 2),
      wft_bf16, bf.reshape(1, _VOCAB))


def kernel(input_ids, emb, W1, b1, Wf, bf):
    wft = Wf.T.astype(jnp.bfloat16)
    half = _BATCH // 2
    p1 = _sc_pool(input_ids[:half], emb)
    p2 = _sc_pool(input_ids[half:], emb)
    l1 = _tc_mlp(p1, W1, b1, wft, bf)
    l2 = _tc_mlp(p2, W1, b1, wft, bf)
    return jnp.concatenate([l1, l2], axis=0).astype(jnp.float32)


# speedup vs baseline: 1.4209x; 1.4209x over previous
import jax
import jax.numpy as jnp
from jax import lax
from jax.experimental import pallas as pl
from jax.experimental.pallas import tpu as pltpu
from jax.experimental.pallas import tpu_sc as plsc

_VOCAB = 100000
_EMBED = 64
_BATCH = 1024
_CTX = 50
_NC = 2
_NS = 16
_NW = _NC * _NS
_FIRE = 8


def _sc_pool(input_ids, emb):
    batch = input_ids.shape[0]
    bpw = batch // _NW

    def body(ids_hbm, emb_hbm, out_hbm, idx_v, rows_v, out_v, sem):
        wid = lax.axis_index("s") * _NC + lax.axis_index("c")
        base = wid * bpw
        pltpu.sync_copy(ids_hbm.at[pl.ds(base, bpw), :], idx_v)

        for g in range(pl.cdiv(bpw, _FIRE)):
            cps = []
            for i in range(min(_FIRE, bpw - g * _FIRE)):
                b = g * _FIRE + i
                cps.append(
                    pltpu.async_copy(emb_hbm.at[idx_v.at[b]], rows_v.at[b],
                                     sem)
                )
            for cp in cps:
                cp.wait()

        def bbody(b, carry):
            accs = [rows_v[b, 0, pl.ds(16 * d, 16)] for d in range(4)]
            for c in range(1, _CTX):
                for d in range(4):
                    accs[d] = accs[d] + rows_v[b, c, pl.ds(16 * d, 16)]
            for d in range(4):
                out_v[b, pl.ds(16 * d, 16)] = accs[d] * (1.0 / _CTX)
            return carry

        lax.fori_loop(0, bpw, bbody, 0)
        pltpu.sync_copy(out_v, out_hbm.at[pl.ds(base, bpw), :])

    mesh = plsc.VectorSubcoreMesh(core_axis_name="c", subcore_axis_name="s")
    return pl.kernel(
        body,
        out_type=jax.ShapeDtypeStruct((batch, _EMBED), jnp.float32),
        mesh=mesh,
        compiler_params=pltpu.CompilerParams(use_tc_tiling_on_sc=False),
        scratch_types=[
            pltpu.VMEM((bpw, _CTX), jnp.int32),
            pltpu.VMEM((bpw, _CTX, _EMBED), jnp.float32),
            pltpu.VMEM((bpw, _EMBED), jnp.float32),
            pltpu.SemaphoreType.DMA,
        ],
    )(input_ids, emb)


_BSUB = 32
_NOBUF = 3
_CROWS = 16
_NCHUNK = _BSUB // _CROWS


def _tc_mlp(pooled, W1, b1, wft_bf16, bf):
    batch = pooled.shape[0]
    nstep = batch // _BSUB

    def body(pooled_ref, w1_ref, b1_ref, wft_ref, bf_ref, out_hbm,
             obuf, sems):
        i = pl.program_id(0)
        slot = lax.rem(i, _NOBUF)

        @pl.when(i >= _NOBUF)
        def _():
            for c in range(_NCHUNK):
                pltpu.make_async_copy(
                    obuf.at[slot, pl.ds(c * _CROWS, _CROWS), :],
                    out_hbm.at[
                        pl.ds((i - _NOBUF) * _BSUB + c * _CROWS, _CROWS), :],
                    sems.at[slot],
                ).wait()

        hidden = jnp.maximum(
            lax.dot_general(
                pooled_ref[...].astype(jnp.bfloat16),
                w1_ref[...].astype(jnp.bfloat16),
                (((1,), (1,)), ((), ())),
                preferred_element_type=jnp.float32,
            ) + b1_ref[...],
            0.0,
        )
        obuf[slot] = (lax.dot_general(
            hidden.astype(jnp.bfloat16), wft_ref[...],
            (((1,), (0,)), ((), ())),
            preferred_element_type=jnp.float32,
        ) + bf_ref[...]).astype(jnp.bfloat16)

        for c in range(_NCHUNK):
            pltpu.make_async_copy(
                obuf.at[slot, pl.ds(c * _CROWS, _CROWS), :],
                out_hbm.at[pl.ds(i * _BSUB + c * _CROWS, _CROWS), :],
                sems.at[slot],
            ).start()

        @pl.when(i == nstep - 1)
        def _():
            for k in range(_NOBUF):
                s = (slot - k) % _NOBUF
                for c in range(_NCHUNK):
                    pltpu.make_async_copy(
                        obuf.at[s, pl.ds(c * _CROWS, _CROWS), :],
                        out_hbm.at[
                            pl.ds((i - k) * _BSUB + c * _CROWS, _CROWS), :],
                        sems.at[s],
                    ).wait()

    return pl.pallas_call(
        body,
        grid=(nstep,),
        in_specs=[
            pl.BlockSpec((_BSUB, _EMBED), lambda i: (i, 0)),
            pl.BlockSpec((_EMBED // 2, _EMBED), lambda i: (0, 0)),
            pl.BlockSpec((1, _EMBED // 2), lambda i: (0, 0)),
            pl.BlockSpec((_EMBED // 2, _VOCAB), lambda i: (0, 0)),
            pl.BlockSpec((1, _VOCAB), lambda i: (0, 0)),
        ],
        out_specs=pl.BlockSpec(memory_space=pl.ANY),
        out_shape=jax.ShapeDtypeStruct((batch, _VOCAB), jnp.bfloat16),
        scratch_shapes=[
            pltpu.VMEM((_NOBUF, _BSUB, _VOCAB), jnp.bfloat16),
            pltpu.SemaphoreType.DMA((_NOBUF,)),
        ],
        compiler_params=pltpu.CompilerParams(
            vmem_limit_bytes=60_000_000,
        ),
    )(pooled, W1, b1.reshape(1, _EMBED // 2),
      wft_bf16, bf.reshape(1, _VOCAB))


def kernel(input_ids, emb, W1, b1, Wf, bf):
    pooled = _sc_pool(input_ids, emb)
    wft = Wf.T.astype(jnp.bfloat16)
    return _tc_mlp(pooled, W1, b1, wft, bf).astype(jnp.float32)
